# baseline (speedup 1.0000x reference)
import functools

import jax
import jax.numpy as jnp
from jax import lax
from jax.experimental import pallas as pl
from jax.experimental.pallas import tpu as pltpu
from jax.experimental.pallas import tpu_sc as plsc

_NC = 2
_NS = 16
_NW = _NC * _NS
_LANES = 16
_CHUNK = 128


def _make_kernel(vocab, dim, batch):
    b_per_w = batch // _NW
    n_chunks = b_per_w // _CHUNK
    groups_per_chunk = _CHUNK // _LANES
    row_w = 128
    per_super = row_w // dim
    mesh = plsc.VectorSubcoreMesh(core_axis_name="c", subcore_axis_name="s")

    @functools.partial(
        pl.kernel,
        out_type=jax.ShapeDtypeStruct((batch,), jnp.float32),
        mesh=mesh,
        compiler_params=pltpu.CompilerParams(needs_layout_passes=False),
        scratch_types=[
            pltpu.VMEM((n_chunks, _CHUNK), jnp.int32),
            pltpu.VMEM((n_chunks, _CHUNK), jnp.int32),
            pltpu.VMEM((b_per_w,), jnp.int32),
            pltpu.VMEM((_CHUNK, row_w), jnp.float32),
            pltpu.VMEM((_CHUNK, row_w), jnp.float32),
            pltpu.VMEM((b_per_w,), jnp.float32),
            pltpu.SemaphoreType.DMA,
        ],
    )
    def k(hero_sup, ctx_sup, off_in, hero_tab, ctx_tab, out,
          hsup_v, csup_v, off_v, hrow_v, crow_v, score_v, sem):
        wid = lax.axis_index("s") * _NC + lax.axis_index("c")
        base = wid * b_per_w

        pltpu.sync_copy(hero_sup.at[wid], hsup_v)
        pltpu.sync_copy(ctx_sup.at[wid], csup_v)
        pltpu.sync_copy(off_in.at[pl.ds(base, b_per_w)], off_v)

        lane = lax.iota(jnp.int32, _LANES)

        def chunk(ch, carry):
            e0 = ch * _CHUNK
            c1 = pltpu.async_copy(hero_tab.at[hsup_v.at[ch]], hrow_v, sem)
            c2 = pltpu.async_copy(ctx_tab.at[csup_v.at[ch]], crow_v, sem)
            c1.wait()
            c2.wait()

            def group(g, carry2):
                ge0 = g * _LANES
                off = off_v[pl.ds(e0 + ge0, _LANES)]
                hoff = off & 0xFFFF
                coff = lax.shift_right_logical(off, 16)
                row = ge0 + lane
                acc = jnp.zeros((_LANES,), jnp.float32)
                for d in range(dim):
                    h = plsc.load_gather(hrow_v, [row, hoff + d])
                    c = plsc.load_gather(crow_v, [row, coff + d])
                    acc = acc + h * c
                score_v[pl.ds(e0 + ge0, _LANES)] = acc
                return carry2

            lax.fori_loop(0, groups_per_chunk, group, 0)
            return carry

        lax.fori_loop(0, n_chunks, chunk, 0)

        pltpu.sync_copy(score_v, out.at[pl.ds(base, b_per_w)])

    return k


@jax.jit
def kernel(hero_ids, context_ids, hero_table, context_table):
    vocab, dim = hero_table.shape
    batch = hero_ids.shape[0]
    b_per_w = batch // _NW
    n_chunks = b_per_w // _CHUNK
    per_super = 128 // dim
    k = _make_kernel(vocab, dim, batch)
    hids = hero_ids.astype(jnp.int32)
    cids = context_ids.astype(jnp.int32)
    hero_sup = (hids // per_super).reshape(_NW, n_chunks, _CHUNK)
    ctx_sup = (cids // per_super).reshape(_NW, n_chunks, _CHUNK)
    off = (hids % per_super) * dim + (((cids % per_super) * dim) << 16)
    hero_wide = hero_table.reshape(vocab // per_super, 128)
    ctx_wide = context_table.reshape(vocab // per_super, 128)
    return k(hero_sup, ctx_sup, off, hero_wide, ctx_wide)
